# baseline (speedup 1.0000x reference)
import jax
import jax.numpy as jnp
from jax.experimental import pallas as pl
from jax.experimental.pallas import tpu as pltpu

B, L, N = 1024, 20, 1000
EMB, U = 64, 128
ALPHA = 0.5
NP = 1024
EP = 128
BE = 128


def _body(x_ref, lenf_ref, a_ref, ib_ref, cb_ref, wr_ref, wenc_ref, benc_ref,
          wcat_ref, bl_ref, wH_ref, out_ref):
    thr = jnp.abs(cb_ref[0, 0])
    h = jnp.zeros((BE, U), jnp.float32)
    c = jnp.zeros((BE, U), jnp.float32)
    hT = jnp.zeros((BE, U), jnp.float32)
    for t in range(L):
        x = x_ref[:, t, :]
        xa = jnp.dot(x, a_ref[:N, :], preferred_element_type=jnp.float32)
        ga = jax.nn.relu(xa - thr)
        emb = jnp.dot(x, wr_ref[...], preferred_element_type=jnp.float32)
        emb += jnp.dot(ga, wenc_ref[...], preferred_element_type=jnp.float32)
        emb = jax.nn.relu(emb + benc_ref[...])
        z = jnp.dot(jnp.concatenate([emb, h], axis=1), wcat_ref[...],
                    preferred_element_type=jnp.float32) + bl_ref[...]
        i = jax.nn.sigmoid(z[:, :U])
        f = jax.nn.sigmoid(z[:, U:2 * U])
        g = jnp.tanh(z[:, 2 * U:3 * U])
        o = jax.nn.sigmoid(z[:, 3 * U:])
        c = f * c + i * g
        h = o * jnp.tanh(c)
        hT = jnp.where(lenf_ref[...] == float(t), h, hT)
    p = jax.nn.sigmoid(
        jnp.dot(hT, wH_ref[...], preferred_element_type=jnp.float32))
    pa = jnp.dot(p, a_ref[...], preferred_element_type=jnp.float32)
    r = jax.nn.relu(ib_ref[...])
    logits = (1.0 - ALPHA) * p + ALPHA * (p * r + jax.nn.relu(pa))
    out_ref[...] = jax.nn.sigmoid(logits)[:, :N]


@jax.jit
def kernel(bseq, bseq_length, A, I_B, C_B, W_enc, b_enc, Wx, Wh, b_lstm, W_H):
    pad = NP - N
    epad = EP - EMB
    A_p = jnp.pad(A, ((0, pad), (0, pad)))
    ib_p = jnp.pad(I_B, (0, pad)).reshape(1, NP)
    W_r = jnp.pad(jax.nn.relu(I_B)[:, None] * W_enc, ((0, 0), (0, epad)))
    W_enc_p = jnp.pad(W_enc, ((0, pad), (0, epad)))
    benc = jnp.pad(b_enc, (0, epad)).reshape(1, EP)
    W_cat = jnp.concatenate([jnp.pad(Wx, ((0, epad), (0, 0))), Wh], axis=0)
    lenf = (bseq_length.astype(jnp.float32) - 1.0).reshape(B, 1)
    cb = C_B.reshape(1, 1)
    bl = b_lstm.reshape(1, 4 * U)
    W_H_p = jnp.pad(W_H, ((0, 0), (0, pad)))

    w = lambda i: (0, 0)
    out = pl.pallas_call(
        _body,
        grid=(B // BE,),
        in_specs=[
            pl.BlockSpec((BE, L, N), lambda i: (i, 0, 0)),
            pl.BlockSpec((BE, 1), lambda i: (i, 0)),
            pl.BlockSpec((NP, NP), w),
            pl.BlockSpec((1, NP), w),
            pl.BlockSpec((1, 1), w),
            pl.BlockSpec((N, EP), w),
            pl.BlockSpec((NP, EP), w),
            pl.BlockSpec((1, EP), w),
            pl.BlockSpec((2 * EP, 4 * U), w),
            pl.BlockSpec((1, 4 * U), w),
            pl.BlockSpec((U, NP), w),
        ],
        out_specs=pl.BlockSpec((BE, N), lambda i: (i, 0)),
        out_shape=jax.ShapeDtypeStruct((B, N), jnp.float32),
        compiler_params=pltpu.CompilerParams(
            vmem_limit_bytes=63 * 1024 * 1024),
    )(bseq, lenf, A_p, ib_p, cb, W_r, W_enc_p, benc, W_cat, bl, W_H_p)
    return out


# speedup vs baseline: 1.0480x; 1.0480x over previous
import jax
import jax.numpy as jnp
from jax.experimental import pallas as pl
from jax.experimental.pallas import tpu as pltpu

B, L, N = 1024, 20, 1000
EMB, U = 64, 128
ALPHA = 0.5
NP = 1024
EP = 128
BE = 128


def _body(x_ref, lenf_ref, a_ref, ib_ref, cb_ref, wr_ref, wenc_ref, benc_ref,
          wcat_ref, bl_ref, wH_ref, out_ref):
    thr = jnp.abs(cb_ref[0, 0])
    h = jnp.zeros((BE, U), jnp.float32)
    c = jnp.zeros((BE, U), jnp.float32)
    hT = jnp.zeros((BE, U), jnp.float32)
    for tp in range(L // 2):
        x2 = jnp.concatenate(
            [x_ref[:, 2 * tp, :], x_ref[:, 2 * tp + 1, :]], axis=0)
        xa2 = jnp.dot(x2, a_ref[:N, :], preferred_element_type=jnp.float32)
        ga2 = jax.nn.relu(xa2 - thr)
        emb2 = jnp.dot(x2, wr_ref[...], preferred_element_type=jnp.float32)
        emb2 += jnp.dot(ga2, wenc_ref[...], preferred_element_type=jnp.float32)
        emb2 = jax.nn.relu(emb2 + benc_ref[...])
        for s_ in range(2):
            t = 2 * tp + s_
            emb = emb2[s_ * BE:(s_ + 1) * BE, :]
            z = jnp.dot(jnp.concatenate([emb, h], axis=1), wcat_ref[...],
                        preferred_element_type=jnp.float32) + bl_ref[...]
            i = jax.nn.sigmoid(z[:, :U])
            f = jax.nn.sigmoid(z[:, U:2 * U])
            g = jnp.tanh(z[:, 2 * U:3 * U])
            o = jax.nn.sigmoid(z[:, 3 * U:])
            c = f * c + i * g
            h = o * jnp.tanh(c)
            hT = jnp.where(lenf_ref[...] == float(t), h, hT)
    p = jax.nn.sigmoid(
        jnp.dot(hT, wH_ref[...], preferred_element_type=jnp.float32))
    pa = jnp.dot(p, a_ref[...], preferred_element_type=jnp.float32)
    r = jax.nn.relu(ib_ref[...])
    logits = (1.0 - ALPHA) * p + ALPHA * (p * r + jax.nn.relu(pa))
    out_ref[...] = jax.nn.sigmoid(logits)[:, :N]


@jax.jit
def kernel(bseq, bseq_length, A, I_B, C_B, W_enc, b_enc, Wx, Wh, b_lstm, W_H):
    pad = NP - N
    epad = EP - EMB
    A_p = jnp.pad(A, ((0, pad), (0, pad)))
    ib_p = jnp.pad(I_B, (0, pad)).reshape(1, NP)
    W_r = jnp.pad(jax.nn.relu(I_B)[:, None] * W_enc, ((0, 0), (0, epad)))
    W_enc_p = jnp.pad(W_enc, ((0, pad), (0, epad)))
    benc = jnp.pad(b_enc, (0, epad)).reshape(1, EP)
    W_cat = jnp.concatenate([jnp.pad(Wx, ((0, epad), (0, 0))), Wh], axis=0)
    lenf = (bseq_length.astype(jnp.float32) - 1.0).reshape(B, 1)
    cb = C_B.reshape(1, 1)
    bl = b_lstm.reshape(1, 4 * U)
    W_H_p = jnp.pad(W_H, ((0, 0), (0, pad)))

    w = lambda i: (0, 0)
    out = pl.pallas_call(
        _body,
        grid=(B // BE,),
        in_specs=[
            pl.BlockSpec((BE, L, N), lambda i: (i, 0, 0)),
            pl.BlockSpec((BE, 1), lambda i: (i, 0)),
            pl.BlockSpec((NP, NP), w),
            pl.BlockSpec((1, NP), w),
            pl.BlockSpec((1, 1), w),
            pl.BlockSpec((N, EP), w),
            pl.BlockSpec((NP, EP), w),
            pl.BlockSpec((1, EP), w),
            pl.BlockSpec((2 * EP, 4 * U), w),
            pl.BlockSpec((1, 4 * U), w),
            pl.BlockSpec((U, NP), w),
        ],
        out_specs=pl.BlockSpec((BE, N), lambda i: (i, 0)),
        out_shape=jax.ShapeDtypeStruct((B, N), jnp.float32),
        compiler_params=pltpu.CompilerParams(
            vmem_limit_bytes=63 * 1024 * 1024),
    )(bseq, lenf, A_p, ib_p, cb, W_r, W_enc_p, benc, W_cat, bl, W_H_p)
    return out


# speedup vs baseline: 1.0539x; 1.0057x over previous
import jax
import jax.numpy as jnp
from jax.experimental import pallas as pl
from jax.experimental.pallas import tpu as pltpu

B, L, N = 1024, 20, 1000
EMB, U = 64, 128
ALPHA = 0.5
NP = 1024
EP = 128
BE = 128


def _body(x_ref, lenf_ref, a_ref, ib_ref, cb_ref, wr_ref, wenc_ref, benc_ref,
          wcat_ref, bl_ref, wH_ref, out_ref):
    thr = jnp.abs(cb_ref[0, 0])
    h = jnp.zeros((BE, U), jnp.float32)
    c = jnp.zeros((BE, U), jnp.float32)
    hT = jnp.zeros((BE, U), jnp.float32)
    for tp in range(L // 4):
        x2 = jnp.concatenate(
            [x_ref[:, 4 * tp + j, :] for j in range(4)], axis=0)
        xa2 = jnp.dot(x2, a_ref[:N, :], preferred_element_type=jnp.float32)
        ga2 = jax.nn.relu(xa2 - thr)
        emb2 = jnp.dot(x2, wr_ref[...], preferred_element_type=jnp.float32)
        emb2 += jnp.dot(ga2, wenc_ref[...], preferred_element_type=jnp.float32)
        emb2 = jax.nn.relu(emb2 + benc_ref[...])
        for s_ in range(4):
            t = 4 * tp + s_
            emb = emb2[s_ * BE:(s_ + 1) * BE, :]
            z = jnp.dot(jnp.concatenate([emb, h], axis=1), wcat_ref[...],
                        preferred_element_type=jnp.float32) + bl_ref[...]
            i = jax.nn.sigmoid(z[:, :U])
            f = jax.nn.sigmoid(z[:, U:2 * U])
            g = jnp.tanh(z[:, 2 * U:3 * U])
            o = jax.nn.sigmoid(z[:, 3 * U:])
            c = f * c + i * g
            h = o * jnp.tanh(c)
            hT = jnp.where(lenf_ref[...] == float(t), h, hT)
    p = jax.nn.sigmoid(
        jnp.dot(hT, wH_ref[...], preferred_element_type=jnp.float32))
    pa = jnp.dot(p, a_ref[...], preferred_element_type=jnp.float32)
    r = jax.nn.relu(ib_ref[...])
    logits = (1.0 - ALPHA) * p + ALPHA * (p * r + jax.nn.relu(pa))
    out_ref[...] = jax.nn.sigmoid(logits)[:, :N]


@jax.jit
def kernel(bseq, bseq_length, A, I_B, C_B, W_enc, b_enc, Wx, Wh, b_lstm, W_H):
    pad = NP - N
    epad = EP - EMB
    A_p = jnp.pad(A, ((0, pad), (0, pad)))
    ib_p = jnp.pad(I_B, (0, pad)).reshape(1, NP)
    W_r = jnp.pad(jax.nn.relu(I_B)[:, None] * W_enc, ((0, 0), (0, epad)))
    W_enc_p = jnp.pad(W_enc, ((0, pad), (0, epad)))
    benc = jnp.pad(b_enc, (0, epad)).reshape(1, EP)
    W_cat = jnp.concatenate([jnp.pad(Wx, ((0, epad), (0, 0))), Wh], axis=0)
    lenf = (bseq_length.astype(jnp.float32) - 1.0).reshape(B, 1)
    cb = C_B.reshape(1, 1)
    bl = b_lstm.reshape(1, 4 * U)
    W_H_p = jnp.pad(W_H, ((0, 0), (0, pad)))

    w = lambda i: (0, 0)
    out = pl.pallas_call(
        _body,
        grid=(B // BE,),
        in_specs=[
            pl.BlockSpec((BE, L, N), lambda i: (i, 0, 0)),
            pl.BlockSpec((BE, 1), lambda i: (i, 0)),
            pl.BlockSpec((NP, NP), w),
            pl.BlockSpec((1, NP), w),
            pl.BlockSpec((1, 1), w),
            pl.BlockSpec((N, EP), w),
            pl.BlockSpec((NP, EP), w),
            pl.BlockSpec((1, EP), w),
            pl.BlockSpec((2 * EP, 4 * U), w),
            pl.BlockSpec((1, 4 * U), w),
            pl.BlockSpec((U, NP), w),
        ],
        out_specs=pl.BlockSpec((BE, N), lambda i: (i, 0)),
        out_shape=jax.ShapeDtypeStruct((B, N), jnp.float32),
        compiler_params=pltpu.CompilerParams(
            vmem_limit_bytes=63 * 1024 * 1024),
    )(bseq, lenf, A_p, ib_p, cb, W_r, W_enc_p, benc, W_cat, bl, W_H_p)
    return out
